# Initial kernel scaffold; baseline (speedup 1.0000x reference)
import jax
import jax.numpy as jnp
from jax.experimental import pallas as pl


def kernel(stu_id, exer_id, W_student, W_k_difficulty, W_e_discrimination):
    raise NotImplementedError("write your pallas kernel here")


# speedup vs baseline: 1.1258x; 1.1258x over previous
import functools
import jax
import jax.numpy as jnp
from jax import lax
from jax.experimental import pallas as pl
from jax.experimental.pallas import tpu as pltpu, tpu_sc as plsc

B = 16384
NC = 2
NS = 16
NW = NC * NS
PER_W = B // NW
CH = 4
CHW = PER_W // CH
L = 16


def _sigmoid(x):
    return 1.0 / (1.0 + jnp.exp(-x))


def _irt_body(stu_hbm, exer_hbm, ws_hbm, wk_hbm, we_hbm, out_hbm,
              si_v, ei_v, s_v, k_v, e_v, o_v, sem):
    wid = lax.axis_index("s") * NC + lax.axis_index("c")

    pltpu.sync_copy(stu_hbm.at[wid], si_v)
    pltpu.sync_copy(exer_hbm.at[wid], ei_v)

    copies = []
    for j in range(CH):
        copies.append(pltpu.async_copy(ws_hbm.at[si_v.at[j]], s_v.at[j], sem))
        copies.append(pltpu.async_copy(wk_hbm.at[ei_v.at[j]], k_v.at[j], sem))
        copies.append(pltpu.async_copy(we_hbm.at[ei_v.at[j]], e_v.at[j], sem))
    for c in copies:
        c.wait()

    for j in range(CH):
        for i in range(CHW // L):
            sl = pl.ds(i * L, L)
            s = _sigmoid(s_v[j, sl])
            k = _sigmoid(k_v[j, sl])
            e = _sigmoid(e_v[j, sl])
            o_v[j, sl] = _sigmoid(1.7 * e * (s - k))

    pltpu.sync_copy(o_v, out_hbm.at[wid])


@jax.jit
def _irt_sc(stu_id, exer_id, ws, wk, we):
    run = pl.kernel(
        _irt_body,
        out_type=jax.ShapeDtypeStruct((NW, CH, CHW), jnp.float32),
        mesh=plsc.VectorSubcoreMesh(core_axis_name="c", subcore_axis_name="s"),
        scratch_types=[
            pltpu.VMEM((CH, CHW), jnp.int32),
            pltpu.VMEM((CH, CHW), jnp.int32),
            pltpu.VMEM((CH, CHW), jnp.float32),
            pltpu.VMEM((CH, CHW), jnp.float32),
            pltpu.VMEM((CH, CHW), jnp.float32),
            pltpu.VMEM((CH, CHW), jnp.float32),
            pltpu.SemaphoreType.DMA,
        ],
    )
    return run(stu_id, exer_id, ws, wk, we)


def kernel(stu_id, exer_id, W_student, W_k_difficulty, W_e_discrimination):
    stu = stu_id.astype(jnp.int32).reshape(NW, CH, CHW)
    exer = exer_id.astype(jnp.int32).reshape(NW, CH, CHW)
    out = _irt_sc(
        stu, exer,
        W_student.reshape(-1),
        W_k_difficulty.reshape(-1),
        W_e_discrimination.reshape(-1),
    )
    return out.reshape(B)
